# Initial kernel scaffold; baseline (speedup 1.0000x reference)
import jax
import jax.numpy as jnp
from jax.experimental import pallas as pl


def kernel(params, log_probs, x_ids, edge_index, edge_attr, nodes_sampled, intra_ei, intra_ea, batch):
    raise NotImplementedError("write your pallas kernel here")


# speedup vs baseline: 1.5625x; 1.5625x over previous
import jax
import jax.numpy as jnp
import numpy as np
from jax.experimental import pallas as pl

_H = 128
_NH = 4
_NB = 16
_MAXD = 32


def _dot(a, b):
    return jnp.dot(a.astype(jnp.bfloat16), b.astype(jnp.bfloat16),
                   preferred_element_type=jnp.float32)


def _mlp2_body(x_ref, w1_ref, b1_ref, w2_ref, b2_ref, o_ref):
    t = jnp.maximum(_dot(x_ref[...], w1_ref[...]) + b1_ref[...], 0.0)
    o_ref[...] = _dot(t, w2_ref[...]) + b2_ref[...]


def _lin_body(x_ref, w_ref, b_ref, o_ref):
    o_ref[...] = _dot(x_ref[...], w_ref[...]) + b_ref[...]


def _pick_blk(n):
    for c in (1024, 1000, 512, 400, 256, 200, 128, 80, 40, 8):
        if n % c == 0:
            return c
    return n


def _mlp2(x, w1, b1, w2, b2):
    n = x.shape[0]
    blk = _pick_blk(n)
    return pl.pallas_call(
        _mlp2_body,
        grid=(n // blk,),
        in_specs=[
            pl.BlockSpec((blk, _H), lambda i: (i, 0)),
            pl.BlockSpec((_H, _H), lambda i: (0, 0)),
            pl.BlockSpec((1, _H), lambda i: (0, 0)),
            pl.BlockSpec((_H, _H), lambda i: (0, 0)),
            pl.BlockSpec((1, _H), lambda i: (0, 0)),
        ],
        out_specs=pl.BlockSpec((blk, _H), lambda i: (i, 0)),
        out_shape=jax.ShapeDtypeStruct((n, _H), jnp.float32),
    )(x, w1, b1.reshape(1, _H), w2, b2.reshape(1, _H))


def _lin(x, w, b):
    n = x.shape[0]
    blk = _pick_blk(n)
    return pl.pallas_call(
        _lin_body,
        grid=(n // blk,),
        in_specs=[
            pl.BlockSpec((blk, _H), lambda i: (i, 0)),
            pl.BlockSpec((_H, _H), lambda i: (0, 0)),
            pl.BlockSpec((1, _H), lambda i: (0, 0)),
        ],
        out_specs=pl.BlockSpec((blk, _H), lambda i: (i, 0)),
        out_shape=jax.ShapeDtypeStruct((n, _H), jnp.float32),
    )(x, w, b.reshape(1, _H))


def _bn(x, g, b):
    mu = x.mean(axis=0)
    var = x.var(axis=0)
    return (x - mu) / jnp.sqrt(var + 1e-5) * g + b


def kernel(params, log_probs, x_ids, edge_index, edge_attr, nodes_sampled,
           intra_ei, intra_ea, batch):
    NT = x_ids.shape[0]
    S = log_probs.shape[0]
    K = nodes_sampled.shape[1]
    SK = S * K
    M = S // NT
    H = _H

    atom = params["atom_emb"][x_ids[:, 0]]
    node_ids = nodes_sampled.reshape(-1)
    valid_f = (node_ids >= 0).astype(jnp.float32)[:, None]
    clamped = jnp.clip(node_ids, 0, None)
    x_flat = atom[clamped]

    src = intra_ei[0]
    dst = intra_ei[1]
    sub = src // K
    so = src % K
    do = dst % K

    A = jnp.zeros((5, S, K, K), jnp.float32).at[intra_ea, sub, so, do].add(1.0)

    Wm = jnp.where(A.sum(0) > 0.0, 1.0, 1e9)
    D = jnp.full((S, K), 64.0, jnp.float32).at[:, 0].set(0.0)
    for _ in range(K):
        cand = jnp.minimum(64.0, (D[:, :, None] + Wm).min(axis=1))
        D = jnp.minimum(D, cand)
    dist = jnp.clip(D, 0.0, float(_MAXD)).astype(jnp.int32)
    dist_pe = params["dist_emb"][dist.reshape(SK)]

    logp_pe = jax.nn.relu(
        log_probs[:, None] @ params["logp_W"] + params["logp_b"])
    h = (x_flat + dist_pe + jnp.repeat(logp_pe, K, axis=0)) * valid_f

    pos = jnp.arange(SK) % K
    is_root_f = (pos == 0).astype(jnp.float32)[:, None]

    esrc = edge_index[0]
    edst = edge_index[1]
    etype = edge_attr[:, 0] - 1

    for lp in params["layers"]:
        ip = lp["intra"]
        ep = lp["inter"]

        T_i = params["bond_emb"][:5] @ ip["eW"] + ip["eb"]
        hS = h.reshape(S, K, H)
        agg = jnp.zeros((S, K, H), jnp.float32)
        for t in range(5):
            agg = agg + jnp.einsum(
                'sij,sih->sjh', A[t], jax.nn.relu(hS + T_i[t]),
                precision=jax.lax.Precision.HIGHEST,
                preferred_element_type=jnp.float32)
        pre = (1.0 + ip["eps"]) * h + agg.reshape(SK, H)
        h1 = _mlp2(pre, ip["W1"], ip["b1"], ip["W2"], ip["b2"])
        h1 = _bn(h1, lp["i_g"], lp["i_b"]) * valid_f

        hs_lin = _lin(h, lp["self_W"], lp["self_b"])
        roots_h = hS[:, 0]
        r_lin = _lin(roots_h, lp["root_W"], lp["root_b"])
        h_nr = hs_lin + jnp.repeat(r_lin, K, axis=0)

        h_canon = roots_h.reshape(NT, M, H).mean(axis=1)

        T_e = params["bond_emb"] @ ep["eW"] + ep["eb"]
        msg = jax.nn.relu(h_canon[esrc] + T_e[etype])
        aggE = jnp.zeros((NT, H), jnp.float32).at[edst].add(msg)
        preE = (1.0 + ep["eps"]) * h_canon + aggE
        hE = _mlp2(preE, ep["W1"], ep["b1"], ep["W2"], ep["b2"])
        hE = _bn(hE, lp["e_g"], lp["e_b"])
        hEb = hE[clamped] * valid_f

        h = is_root_f * (h1 + hEb) + (1.0 - is_root_f) * (h1 + h_nr)
        h = jax.nn.relu(h) * valid_f

    h_sub = h.reshape(S, K, H).sum(axis=1)
    h2 = h_sub.reshape(NT, M, H)

    mp = params["mha"]
    x2 = h2.reshape(NT * M, H)
    dh = H // _NH

    def _split(t):
        return jnp.transpose(t.reshape(NT, M, _NH, dh), (0, 2, 1, 3))

    q = _split(_lin(x2, mp["Wq"], mp["bq"]))
    k = _split(_lin(x2, mp["Wk"], mp["bk"]))
    v = _split(_lin(x2, mp["Wv"], mp["bv"]))
    a = jax.nn.softmax(q @ jnp.transpose(k, (0, 1, 3, 2)) / np.sqrt(dh),
                       axis=-1)
    o = jnp.transpose(a @ v, (0, 2, 1, 3)).reshape(NT * M, H)
    h_attn = (_lin(o, mp["Wo"], mp["bo"])).reshape(NT, M, H) + h2

    node_emb = h_attn.mean(axis=1)
    node_emb = _bn(node_emb, params["ro_g"], params["ro_b"])
    out = jnp.zeros((_NB, H), jnp.float32).at[batch].add(node_emb)
    return out
